# Initial kernel scaffold; baseline (speedup 1.0000x reference)
import jax
import jax.numpy as jnp
from jax.experimental import pallas as pl


def kernel(X, Z):
    raise NotImplementedError("write your pallas kernel here")


# speedup vs baseline: 32.9191x; 32.9191x over previous
import functools

import jax
import jax.numpy as jnp
from jax import lax
from jax.experimental import pallas as pl

K = 10
REG = 1e-06
BLK = 128


def _lle_block(x_ref, z_ref, out_ref):
    i = pl.program_id(0)
    nblk = pl.num_programs(0)
    X = x_ref[...]
    Z = z_ref[...]
    N = X.shape[0]
    xb = x_ref[pl.ds(i * BLK, BLK), :]
    zb = z_ref[pl.ds(i * BLK, BLK), :]

    G = lax.dot_general(xb, X, (((1,), (1,)), ((), ())),
                        preferred_element_type=jnp.float32)
    sq_all = jnp.sum(X * X, axis=1)[None, :]
    sq_b = jnp.sum(xb * xb, axis=1)[:, None]
    D2 = sq_b + sq_all - 2.0 * G

    col = lax.broadcasted_iota(jnp.int32, (BLK, N), 1)
    vals = D2
    nbrs = []
    for t in range(K + 1):
        m = jnp.min(vals, axis=1, keepdims=True)
        sel = jnp.min(jnp.where(vals == m, col, N), axis=1, keepdims=True)
        if t > 0:
            nbrs.append(sel)
        vals = jnp.where(col == sel, jnp.inf, vals)

    diffs = []
    zn = []
    for a in range(K):
        onehot = (col == nbrs[a]).astype(jnp.float32)
        xn_a = lax.dot_general(onehot, X, (((1,), (0,)), ((), ())),
                               preferred_element_type=jnp.float32)
        zn_a = lax.dot_general(onehot, Z, (((1,), (0,)), ((), ())),
                               preferred_element_type=jnp.float32)
        diffs.append(xn_a - xb)
        zn.append(zn_a)

    ent = {}
    for a in range(K):
        for b in range(a, K):
            cab = jnp.sum(diffs[a] * diffs[b], axis=1, keepdims=True)
            if a == b:
                cab = cab + REG
            ent[(a, b)] = cab
            ent[(b, a)] = cab
    rows = [jnp.concatenate([ent[(a, b)] for b in range(K)], axis=1)
            for a in range(K)]
    rhs = [jnp.ones((BLK, 1), jnp.float32) for _ in range(K)]

    for j in range(K):
        inv = 1.0 / rows[j][:, j:j + 1]
        for r in range(K):
            if r == j:
                continue
            f = rows[r][:, j:j + 1] * inv
            rows[r] = rows[r] - f * rows[j]
            rhs[r] = rhs[r] - f * rhs[j]
    w = [rhs[a] / rows[a][:, a:a + 1] for a in range(K)]
    wsum = functools.reduce(lambda p, q: p + q, w)
    recon = functools.reduce(
        lambda p, q: p + q, [(w[a] / wsum) * zn[a] for a in range(K)])

    partial = jnp.sum((recon - zb) ** 2).reshape(1, 1)

    @pl.when(i == 0)
    def _init():
        out_ref[...] = jnp.zeros((1, 1), jnp.float32)

    acc = out_ref[...] + partial

    @pl.when(i < nblk - 1)
    def _acc():
        out_ref[...] = acc

    @pl.when(i == nblk - 1)
    def _fin():
        out_ref[...] = acc / (N * Z.shape[1])


def kernel(X, Z):
    n = X.shape[0]
    out = pl.pallas_call(
        _lle_block,
        grid=(n // BLK,),
        out_shape=jax.ShapeDtypeStruct((1, 1), jnp.float32),
    )(X, Z)
    return out.reshape(())


# speedup vs baseline: 52.9251x; 1.6077x over previous
import functools

import jax
import jax.numpy as jnp
from jax import lax
from jax.experimental import pallas as pl

K = 10
REG = 1e-06
BLK = 128


def _lle_block(x_ref, z_ref, out_ref):
    i = pl.program_id(0)
    nblk = pl.num_programs(0)
    X = x_ref[...]
    Z = z_ref[...]
    N = X.shape[0]
    xb = x_ref[pl.ds(i * BLK, BLK), :]
    zb = z_ref[pl.ds(i * BLK, BLK), :]

    G = lax.dot_general(xb, X, (((1,), (1,)), ((), ())),
                        preferred_element_type=jnp.float32)
    sq_all = jnp.sum(X * X, axis=1)[None, :]
    sq_b = jnp.sum(xb * xb, axis=1)[:, None]
    D2 = jnp.maximum(sq_b + sq_all - 2.0 * G, 0.0)

    col = lax.broadcasted_iota(jnp.int32, (BLK, N), 1)
    key = (lax.bitcast_convert_type(D2, jnp.int32) & (-N)) | col
    imax = jnp.iinfo(jnp.int32).max
    nbrs = []
    for t in range(K + 1):
        m = jnp.min(key, axis=1, keepdims=True)
        if t > 0:
            nbrs.append(m & (N - 1))
        key = jnp.where(key == m, imax, key)

    diffs = []
    zn = []
    for a in range(K):
        onehot = (col == nbrs[a]).astype(jnp.float32)
        xn_a = lax.dot_general(onehot, X, (((1,), (0,)), ((), ())),
                               preferred_element_type=jnp.float32)
        zn_a = lax.dot_general(onehot, Z, (((1,), (0,)), ((), ())),
                               preferred_element_type=jnp.float32)
        diffs.append(xn_a - xb)
        zn.append(zn_a)

    ent = {}
    for a in range(K):
        for b in range(a, K):
            cab = jnp.sum(diffs[a] * diffs[b], axis=1, keepdims=True)
            if a == b:
                cab = cab + REG
            ent[(a, b)] = cab
            ent[(b, a)] = cab
    rows = [jnp.transpose(
        jnp.concatenate([ent[(a, b)] for b in range(K)], axis=1))
        for a in range(K)]
    rhs = [jnp.ones((1, BLK), jnp.float32) for _ in range(K)]

    for j in range(K):
        inv = 1.0 / rows[j][j:j + 1, :]
        for r in range(K):
            if r == j:
                continue
            f = rows[r][j:j + 1, :] * inv
            rows[r] = rows[r] - f * rows[j]
            rhs[r] = rhs[r] - f * rhs[j]
    w = [rhs[a] / rows[a][a:a + 1, :] for a in range(K)]
    wsum = functools.reduce(lambda p, q: p + q, w)
    wt = jnp.transpose(
        jnp.concatenate([w[a] / wsum for a in range(K)], axis=0))
    recon = functools.reduce(
        lambda p, q: p + q, [wt[:, a:a + 1] * zn[a] for a in range(K)])

    partial = jnp.sum((recon - zb) ** 2).reshape(1, 1)

    @pl.when(i == 0)
    def _init():
        out_ref[...] = jnp.zeros((1, 1), jnp.float32)

    acc = out_ref[...] + partial

    @pl.when(i < nblk - 1)
    def _acc():
        out_ref[...] = acc

    @pl.when(i == nblk - 1)
    def _fin():
        out_ref[...] = acc / (N * Z.shape[1])


def kernel(X, Z):
    n = X.shape[0]
    out = pl.pallas_call(
        _lle_block,
        grid=(n // BLK,),
        out_shape=jax.ShapeDtypeStruct((1, 1), jnp.float32),
    )(X, Z)
    return out.reshape(())
